# baseline (speedup 1.0000x reference)
import jax
import jax.numpy as jnp
from jax.experimental import pallas as pl
from jax.experimental.pallas import tpu as pltpu

B, T, N, H, C = 4, 16, 100, 8, 256
K1 = 5
N_HEADS = 8
BETA = 0.8
BT_BLK = 4
NR = N * N_HEADS
MC = N * H + H


def _s_builder(nn_rep_ref, agg_w_ref, agg_b_ref, s_ref):
    col = jax.lax.broadcasted_iota(jnp.int32, (1, MC), 1)
    m_row = col // H
    acc = jnp.zeros((NR, MC), dtype=jnp.float32)
    for k in range(K1):
        nnk = nn_rep_ref[:, k][:, None]
        wk = agg_w_ref[:, k * H : (k + 1) * H]
        wt = jnp.broadcast_to(
            jnp.tile(wk, (1, MC // H)).reshape(1, N_HEADS, MC),
            (N, N_HEADS, MC),
        ).reshape(NR, MC)
        acc = acc + jnp.where(nnk == m_row, wt, 0.0)
    acc = jnp.where(col == N * H, agg_b_ref[...], acc)
    s_ref[...] = acc.astype(jnp.bfloat16)


def _sage_kernel(x_ref, s_ref, out_w_ref, out_b_ref, o_ref, xe_ref):
    @pl.when(pl.program_id(0) == 0)
    def _init_ones():
        for j in range(BT_BLK):
            xe_ref[j, N * H :, :] = jnp.ones((H, C), dtype=jnp.bfloat16)

    for j in range(BT_BLK):
        xe_ref[j, : N * H, :] = x_ref[j].reshape(N * H, C).astype(jnp.bfloat16)

    accs = [
        jax.lax.dot_general(
            s_ref[...], xe_ref[j],
            dimension_numbers=(((1,), (0,)), ((), ())),
            preferred_element_type=jnp.float32,
        )
        for j in range(BT_BLK)
    ]
    acts = [
        (acc * jax.nn.sigmoid(BETA * acc)).astype(jnp.bfloat16)
        for acc in accs
    ]
    for j in range(BT_BLK):
        out = jax.lax.dot_general(
            acts[j], out_w_ref[...],
            dimension_numbers=(((1,), (1,)), ((), ())),
            preferred_element_type=jnp.float32,
        )
        out = out + out_b_ref[...]
        o_ref[j] = out.reshape(N, N_HEADS, C)


@jax.jit
def _run(x, nearest_nodes, agg_W, agg_b, out_W, out_b):
    bt = B * T
    xr = x.reshape(bt, N, H, C)

    nn_rep = jnp.repeat(nearest_nodes, N_HEADS, axis=0)
    agg_b_t = jnp.tile(agg_b, (N,)).reshape(NR, 1)
    out_w = out_W.astype(jnp.bfloat16)
    out_b2 = out_b.reshape(1, C)

    s = pl.pallas_call(
        _s_builder,
        grid=(1,),
        in_specs=[
            pl.BlockSpec((NR, K1), lambda i: (0, 0)),
            pl.BlockSpec((N_HEADS, K1 * H), lambda i: (0, 0)),
            pl.BlockSpec((NR, 1), lambda i: (0, 0)),
        ],
        out_specs=pl.BlockSpec((NR, MC), lambda i: (0, 0)),
        out_shape=jax.ShapeDtypeStruct((NR, MC), jnp.bfloat16),
    )(nn_rep, agg_W, agg_b_t)

    out = pl.pallas_call(
        _sage_kernel,
        grid=(bt // BT_BLK,),
        in_specs=[
            pl.BlockSpec((BT_BLK, N, H, C), lambda i: (i, 0, 0, 0)),
            pl.BlockSpec((NR, MC), lambda i: (0, 0)),
            pl.BlockSpec((C, C), lambda i: (0, 0)),
            pl.BlockSpec((1, C), lambda i: (0, 0)),
        ],
        out_specs=pl.BlockSpec((BT_BLK, N, H, C), lambda i: (i, 0, 0, 0)),
        out_shape=jax.ShapeDtypeStruct((bt, N, H, C), jnp.float32),
        scratch_shapes=[pltpu.VMEM((BT_BLK, N * H + H, C), jnp.bfloat16)],
    )(xr, s, out_w, out_b2)
    return out.reshape(B, T, N, H, C)


def kernel(x, nearest_nodes, agg_W, agg_b, out_W, out_b):
    return _run(x, nearest_nodes, agg_W, agg_b, out_W, out_b)


# speedup vs baseline: 1.0763x; 1.0763x over previous
import jax
import jax.numpy as jnp
from jax.experimental import pallas as pl
from jax.experimental.pallas import tpu as pltpu

B, T, N, H, C = 4, 16, 100, 8, 256
K1 = 5
N_HEADS = 8
BETA = 0.8
BT_BLK = 4
NR = N * N_HEADS
MC = N * H + H


def _sage_kernel(x_ref, nn_rep_ref, agg_w_ref, agg_b_ref, out_w_ref,
                 out_b_ref, o_ref, s_ref, w2_ref, xe_ref):
    @pl.when(pl.program_id(0) == 0)
    def _build_s():
        col = jax.lax.broadcasted_iota(jnp.int32, (1, MC), 1)
        m_row = col // H
        acc = jnp.zeros((NR, MC), dtype=jnp.float32)
        for k in range(K1):
            nnk = nn_rep_ref[:, k][:, None]
            wk = agg_w_ref[:, k * H : (k + 1) * H]
            wt = jnp.broadcast_to(
                jnp.tile(wk, (1, MC // H)).reshape(1, N_HEADS, MC),
                (N, N_HEADS, MC),
            ).reshape(NR, MC)
            acc = acc + jnp.where(nnk == m_row, wt, 0.0)
        acc = jnp.where(col == N * H, agg_b_ref[...], acc)
        s_ref[...] = acc.astype(jnp.bfloat16)
        w2_ref[...] = out_w_ref[...].astype(jnp.bfloat16)
        for j in range(BT_BLK):
            xe_ref[j, N * H :, :] = jnp.ones((H, C), dtype=jnp.bfloat16)

    for j in range(BT_BLK):
        xe_ref[j, : N * H, :] = x_ref[j].reshape(N * H, C).astype(jnp.bfloat16)

    accs = [
        jax.lax.dot_general(
            s_ref[...], xe_ref[j],
            dimension_numbers=(((1,), (0,)), ((), ())),
            preferred_element_type=jnp.float32,
        )
        for j in range(BT_BLK)
    ]
    acts = [
        (acc * jax.nn.sigmoid(BETA * acc)).astype(jnp.bfloat16)
        for acc in accs
    ]
    for j in range(BT_BLK):
        out = jax.lax.dot_general(
            acts[j], w2_ref[...],
            dimension_numbers=(((1,), (1,)), ((), ())),
            preferred_element_type=jnp.float32,
        )
        out = out + out_b_ref[...]
        o_ref[j] = out.reshape(N, N_HEADS, C)


@jax.jit
def _run(x, nearest_nodes, agg_W, agg_b, out_W, out_b):
    bt = B * T
    xr = x.reshape(bt, N, H, C)

    nn_rep = jnp.repeat(nearest_nodes, N_HEADS, axis=0)
    agg_b_t = jnp.tile(agg_b, (N,)).reshape(NR, 1)
    out_b2 = out_b.reshape(1, C)

    out = pl.pallas_call(
        _sage_kernel,
        grid=(bt // BT_BLK,),
        in_specs=[
            pl.BlockSpec((BT_BLK, N, H, C), lambda i: (i, 0, 0, 0)),
            pl.BlockSpec((NR, K1), lambda i: (0, 0)),
            pl.BlockSpec((N_HEADS, K1 * H), lambda i: (0, 0)),
            pl.BlockSpec((NR, 1), lambda i: (0, 0)),
            pl.BlockSpec((C, C), lambda i: (0, 0)),
            pl.BlockSpec((1, C), lambda i: (0, 0)),
        ],
        out_specs=pl.BlockSpec((BT_BLK, N, H, C), lambda i: (i, 0, 0, 0)),
        out_shape=jax.ShapeDtypeStruct((bt, N, H, C), jnp.float32),
        scratch_shapes=[
            pltpu.VMEM((NR, MC), jnp.bfloat16),
            pltpu.VMEM((C, C), jnp.bfloat16),
            pltpu.VMEM((BT_BLK, N * H + H, C), jnp.bfloat16),
        ],
    )(xr, nn_rep, agg_W, agg_b_t, out_W, out_b2)
    return out.reshape(B, T, N, H, C)


def kernel(x, nearest_nodes, agg_W, agg_b, out_W, out_b):
    return _run(x, nearest_nodes, agg_W, agg_b, out_W, out_b)


# speedup vs baseline: 1.0910x; 1.0136x over previous
import jax
import jax.numpy as jnp
from jax.experimental import pallas as pl
from jax.experimental.pallas import tpu as pltpu

B, T, N, H, C = 4, 16, 100, 8, 256
K1 = 5
N_HEADS = 8
BETA = 0.8
BT_BLK = 4
NR = N * N_HEADS
MC = N * H + H


def _sage_kernel(x_ref, nn_ref, agg_w_ref, agg_b_ref, out_w_ref,
                 out_b_ref, o_ref, s_ref, w2_ref, xe_ref):
    @pl.when(pl.program_id(0) == 0)
    def _build_s():
        col = jax.lax.broadcasted_iota(jnp.int32, (1, 1, MC), 2)
        m_row = col // H
        acc = jnp.zeros((N, N_HEADS, MC), dtype=jnp.float32)
        for k in range(K1):
            nnk = nn_ref[:, :, k][:, :, None]
            wk = agg_w_ref[:, k * H : (k + 1) * H]
            wt = jnp.tile(wk, (1, MC // H)).reshape(1, N_HEADS, MC)
            acc = acc + jnp.where(nnk == m_row, wt, 0.0)
        acc = jnp.where(col == N * H, agg_b_ref[...][None], acc)
        s_ref[...] = acc.reshape(NR, MC).astype(jnp.bfloat16)
        w2_ref[...] = out_w_ref[...].astype(jnp.bfloat16)
        for j in range(BT_BLK):
            xe_ref[j, N * H :, :] = jnp.ones((H, C), dtype=jnp.bfloat16)

    for j in range(BT_BLK):
        xe_ref[j, : N * H, :] = x_ref[j].reshape(N * H, C).astype(jnp.bfloat16)

    accs = [
        jax.lax.dot_general(
            s_ref[...], xe_ref[j],
            dimension_numbers=(((1,), (0,)), ((), ())),
            preferred_element_type=jnp.float32,
        )
        for j in range(BT_BLK)
    ]
    acts = [
        (acc * jax.nn.sigmoid(BETA * acc)).astype(jnp.bfloat16)
        for acc in accs
    ]
    for j in range(BT_BLK):
        out = jax.lax.dot_general(
            acts[j], w2_ref[...],
            dimension_numbers=(((1,), (1,)), ((), ())),
            preferred_element_type=jnp.float32,
        )
        out = out + out_b_ref[...]
        o_ref[j] = out.reshape(N, N_HEADS, C)


@jax.jit
def _run(x, nearest_nodes, agg_W, agg_b, out_W, out_b):
    bt = B * T
    xr = x.reshape(bt, N, H, C)

    nn3 = nearest_nodes.reshape(N, 1, K1)
    agg_b2 = agg_b.reshape(N_HEADS, 1)
    out_b2 = out_b.reshape(1, C)

    out = pl.pallas_call(
        _sage_kernel,
        grid=(bt // BT_BLK,),
        in_specs=[
            pl.BlockSpec((BT_BLK, N, H, C), lambda i: (i, 0, 0, 0)),
            pl.BlockSpec((N, 1, K1), lambda i: (0, 0, 0)),
            pl.BlockSpec((N_HEADS, K1 * H), lambda i: (0, 0)),
            pl.BlockSpec((N_HEADS, 1), lambda i: (0, 0)),
            pl.BlockSpec((C, C), lambda i: (0, 0)),
            pl.BlockSpec((1, C), lambda i: (0, 0)),
        ],
        out_specs=pl.BlockSpec((BT_BLK, N, H, C), lambda i: (i, 0, 0, 0)),
        out_shape=jax.ShapeDtypeStruct((bt, N, H, C), jnp.float32),
        scratch_shapes=[
            pltpu.VMEM((NR, MC), jnp.bfloat16),
            pltpu.VMEM((C, C), jnp.bfloat16),
            pltpu.VMEM((BT_BLK, N * H + H, C), jnp.bfloat16),
        ],
    )(xr, nn3, agg_W, agg_b2, out_W, out_b2)
    return out.reshape(B, T, N, H, C)


def kernel(x, nearest_nodes, agg_W, agg_b, out_W, out_b):
    return _run(x, nearest_nodes, agg_W, agg_b, out_W, out_b)
